# baseline (speedup 1.0000x reference)
import jax
import jax.numpy as jnp
from jax import lax
from jax.experimental import pallas as pl
from jax.experimental.pallas import tpu as pltpu

_B, _C, _H, _W = 8, 64, 32, 32
_K = 8192
_N = _B * _H * _W
_HW = _H * _W
_INV_T = 10.0

_BN = 256
_NPB = _HW // _BN
_KT = 2048


def _rows_body(xf_ref, code_ref, dist_ref, q_ref, s2_ref, cnta_ref):
    @pl.when(pl.program_id(0) == 0)
    def _prologue():
        cbt = code_ref[...]
        s = jnp.sum(cbt * cbt, axis=0, keepdims=True)
        cnt = cbt / jnp.maximum(jnp.sqrt(s), 1e-12)
        csqt = jnp.sum(cnt * cnt, axis=0, keepdims=True)
        cnta_ref[...] = jnp.concatenate(
            [cnt, jnp.ones((1, _K), jnp.float32), csqt], axis=0)

    ft = xf_ref[0]
    s = jnp.sum(ft * ft, axis=0, keepdims=True)
    r = 1.0 / jnp.maximum(jnp.sqrt(s), 1e-12)
    fnt = ft * r
    fnt2 = fnt + fnt
    fsqt = jnp.sum(fnt * fnt, axis=0, keepdims=True)
    ones_n = jnp.ones((1, _BN), jnp.float32)

    m = lax.dot_general(cnta_ref[0:64, :], fnt2, (((0,), (0,)), ((), ())),
                        precision=lax.Precision.HIGHEST,
                        preferred_element_type=jnp.float32)

    fa_t = jnp.concatenate([-fnt2, fsqt, ones_n], axis=0)
    dist_ref[...] = lax.dot_general(jnp.transpose(fa_t), cnta_ref[...],
                                    (((1,), (0,)), ((), ())),
                                    preferred_element_type=jnp.float32)

    e2 = jnp.exp(_INV_T * m)
    s2_ref[...] = jnp.sum(e2, axis=0, keepdims=True)[None]

    e1 = jnp.exp(m)
    qs = lax.dot_general(cnta_ref[0:65, :], e1, (((1,), (0,)), ((), ())),
                         preferred_element_type=jnp.float32)
    q_ref[...] = (qs[0:64, :] / qs[64:65, :])[None]


def _slabs_body(xf_ref, code_ref, s2_ref, assign_ref, cn20t_ref, fnt_ref):
    b = pl.program_id(0)
    j = pl.program_id(1)

    @pl.when(jnp.logical_and(b == 0, j == 0))
    def _prologue():
        cbt = code_ref[...]
        s = jnp.sum(cbt * cbt, axis=0, keepdims=True)
        cn20t_ref[...] = cbt * ((2.0 * _INV_T) / jnp.maximum(jnp.sqrt(s), 1e-12))

    @pl.when(j == 0)
    def _per_batch():
        ft = xf_ref[0]
        s = jnp.sum(ft * ft, axis=0, keepdims=True)
        fnt_ref[...] = ft / jnp.maximum(jnp.sqrt(s), 1e-12)

    arg = lax.dot_general(cn20t_ref[:, pl.ds(j * _KT, _KT)], fnt_ref[...],
                          (((0,), (0,)), ((), ())),
                          precision=lax.Precision.HIGHEST,
                          preferred_element_type=jnp.float32)
    e2 = jnp.exp(arg)
    assign_ref[...] = (e2 / s2_ref[0])[None]


def kernel(x, codebook, cur_iter):
    del cur_iter
    xf = x.reshape(_B, _C, _HW)
    cbt = jnp.transpose(codebook)

    dist, q_t, s2 = pl.pallas_call(
        _rows_body,
        grid=(_N // _BN,),
        in_specs=[
            pl.BlockSpec((1, _C, _BN), lambda i: (i // _NPB, 0, i % _NPB)),
            pl.BlockSpec((_C, _K), lambda i: (0, 0)),
        ],
        out_specs=[
            pl.BlockSpec((_BN, _K), lambda i: (i, 0)),
            pl.BlockSpec((1, _C, _BN), lambda i: (i // _NPB, 0, i % _NPB)),
            pl.BlockSpec((1, 1, _BN), lambda i: (i // _NPB, 0, i % _NPB)),
        ],
        out_shape=[
            jax.ShapeDtypeStruct((_N, _K), jnp.float32),
            jax.ShapeDtypeStruct((_B, _C, _HW), jnp.float32),
            jax.ShapeDtypeStruct((_B, 1, _HW), jnp.float32),
        ],
        scratch_shapes=[
            pltpu.VMEM((66, _K), jnp.float32),
        ],
    )(xf, cbt)

    assign_t = pl.pallas_call(
        _slabs_body,
        grid=(_B, _K // _KT),
        in_specs=[
            pl.BlockSpec((1, _C, _HW), lambda b, j: (b, 0, 0)),
            pl.BlockSpec((_C, _K), lambda b, j: (0, 0)),
            pl.BlockSpec((1, 1, _HW), lambda b, j: (b, 0, 0)),
        ],
        out_specs=pl.BlockSpec((1, _KT, _HW), lambda b, j: (b, j, 0)),
        out_shape=jax.ShapeDtypeStruct((_B, _K, _HW), jnp.float32),
        scratch_shapes=[
            pltpu.VMEM((_C, _K), jnp.float32),
            pltpu.VMEM((_C, _HW), jnp.float32),
        ],
    )(xf, cbt, s2)

    q_feat = q_t.reshape(_B, _C, _H, _W)
    assignment = assign_t.reshape(_B, _K, _H, _W)
    return q_feat, assignment, dist


# speedup vs baseline: 1.6406x; 1.6406x over previous
import jax
import jax.numpy as jnp
from jax import lax
from jax.experimental import pallas as pl
from jax.experimental.pallas import tpu as pltpu

_B, _C, _H, _W = 8, 64, 32, 32
_K = 8192
_N = _B * _H * _W
_HW = _H * _W
_INV_T = 10.0

_BN = 256
_NPB = _HW // _BN
_KT = 2048


def _rows_body(xf_ref, code_ref, dist_ref, q_ref, s2_ref, cnta_ref):
    @pl.when(pl.program_id(0) == 0)
    def _prologue():
        cbt = code_ref[...]
        s = jnp.sum(cbt * cbt, axis=0, keepdims=True)
        cnt = cbt / jnp.maximum(jnp.sqrt(s), 1e-12)
        csqt = jnp.sum(cnt * cnt, axis=0, keepdims=True)
        cnta_ref[...] = jnp.concatenate(
            [cnt, jnp.ones((1, _K), jnp.float32), csqt], axis=0)

    ft = xf_ref[0]
    s = jnp.sum(ft * ft, axis=0, keepdims=True)
    r = 1.0 / jnp.maximum(jnp.sqrt(s), 1e-12)
    fnt = ft * r
    fnt2 = fnt + fnt
    fsqt = jnp.sum(fnt * fnt, axis=0, keepdims=True)
    ones_n = jnp.ones((1, _BN), jnp.float32)

    m = lax.dot_general(cnta_ref[0:64, :], fnt2, (((0,), (0,)), ((), ())),
                        preferred_element_type=jnp.float32)

    fa_t = jnp.concatenate([-fnt2, fsqt, ones_n], axis=0)
    dist_ref[...] = lax.dot_general(jnp.transpose(fa_t), cnta_ref[...],
                                    (((1,), (0,)), ((), ())),
                                    preferred_element_type=jnp.float32)

    e2 = jnp.exp(_INV_T * m)
    s2_ref[...] = jnp.sum(e2, axis=0, keepdims=True)[None]

    e1 = jnp.exp(m)
    qs = lax.dot_general(cnta_ref[0:65, :], e1, (((1,), (0,)), ((), ())),
                         preferred_element_type=jnp.float32)
    q_ref[...] = (qs[0:64, :] / qs[64:65, :])[None]


def _slabs_body(xf_ref, code_ref, s2_ref, assign_ref, cn20t_ref, fnt_ref):
    b = pl.program_id(0)
    j = pl.program_id(1)

    @pl.when(jnp.logical_and(b == 0, j == 0))
    def _prologue():
        cbt = code_ref[...]
        s = jnp.sum(cbt * cbt, axis=0, keepdims=True)
        cn20t_ref[...] = cbt * ((2.0 * _INV_T) / jnp.maximum(jnp.sqrt(s), 1e-12))

    @pl.when(j == 0)
    def _per_batch():
        ft = xf_ref[0]
        s = jnp.sum(ft * ft, axis=0, keepdims=True)
        fnt_ref[...] = ft / jnp.maximum(jnp.sqrt(s), 1e-12)

    arg = lax.dot_general(cn20t_ref[:, pl.ds(j * _KT, _KT)], fnt_ref[...],
                          (((0,), (0,)), ((), ())),
                          preferred_element_type=jnp.float32)
    e2 = jnp.exp(arg)
    assign_ref[...] = (e2 / s2_ref[0])[None]


def kernel(x, codebook, cur_iter):
    del cur_iter
    xf = x.reshape(_B, _C, _HW)
    cbt = jnp.transpose(codebook)

    dist, q_t, s2 = pl.pallas_call(
        _rows_body,
        grid=(_N // _BN,),
        in_specs=[
            pl.BlockSpec((1, _C, _BN), lambda i: (i // _NPB, 0, i % _NPB)),
            pl.BlockSpec((_C, _K), lambda i: (0, 0)),
        ],
        out_specs=[
            pl.BlockSpec((_BN, _K), lambda i: (i, 0)),
            pl.BlockSpec((1, _C, _BN), lambda i: (i // _NPB, 0, i % _NPB)),
            pl.BlockSpec((1, 1, _BN), lambda i: (i // _NPB, 0, i % _NPB)),
        ],
        out_shape=[
            jax.ShapeDtypeStruct((_N, _K), jnp.float32),
            jax.ShapeDtypeStruct((_B, _C, _HW), jnp.float32),
            jax.ShapeDtypeStruct((_B, 1, _HW), jnp.float32),
        ],
        scratch_shapes=[
            pltpu.VMEM((66, _K), jnp.float32),
        ],
    )(xf, cbt)

    assign_t = pl.pallas_call(
        _slabs_body,
        grid=(_B, _K // _KT),
        in_specs=[
            pl.BlockSpec((1, _C, _HW), lambda b, j: (b, 0, 0)),
            pl.BlockSpec((_C, _K), lambda b, j: (0, 0)),
            pl.BlockSpec((1, 1, _HW), lambda b, j: (b, 0, 0)),
        ],
        out_specs=pl.BlockSpec((1, _KT, _HW), lambda b, j: (b, j, 0)),
        out_shape=jax.ShapeDtypeStruct((_B, _K, _HW), jnp.float32),
        scratch_shapes=[
            pltpu.VMEM((_C, _K), jnp.float32),
            pltpu.VMEM((_C, _HW), jnp.float32),
        ],
    )(xf, cbt, s2)

    q_feat = q_t.reshape(_B, _C, _H, _W)
    assignment = assign_t.reshape(_B, _K, _H, _W)
    return q_feat, assignment, dist


# speedup vs baseline: 1.8179x; 1.1080x over previous
import jax
import jax.numpy as jnp
from jax import lax
from jax.experimental import pallas as pl
from jax.experimental.pallas import tpu as pltpu

_B, _C, _H, _W = 8, 64, 32, 32
_K = 8192
_N = _B * _H * _W
_HW = _H * _W
_INV_T = 10.0

_BN = 128
_NPB = _HW // _BN


def _vq_body(xf_ref, code_ref, dist_ref, assign_ref, q_ref, cnta_ref):
    @pl.when(pl.program_id(0) == 0)
    def _prologue():
        cbt = code_ref[...]
        s = jnp.sum(cbt * cbt, axis=0, keepdims=True)
        cnt = cbt / jnp.maximum(jnp.sqrt(s), 1e-12)
        csqt = jnp.sum(cnt * cnt, axis=0, keepdims=True)
        cnta_ref[...] = jnp.concatenate(
            [cnt, jnp.ones((1, _K), jnp.float32), csqt], axis=0)

    ft = xf_ref[0]
    s = jnp.sum(ft * ft, axis=0, keepdims=True)
    r = 1.0 / jnp.maximum(jnp.sqrt(s), 1e-12)
    fnt = ft * r
    fnt2 = fnt + fnt
    fsqt = jnp.sum(fnt * fnt, axis=0, keepdims=True)
    ones_n = jnp.ones((1, _BN), jnp.float32)

    m = lax.dot_general(cnta_ref[0:64, :], fnt2, (((0,), (0,)), ((), ())),
                        preferred_element_type=jnp.float32)

    fa_t = jnp.concatenate([-fnt2, fsqt, ones_n], axis=0)
    dist_ref[...] = lax.dot_general(jnp.transpose(fa_t), cnta_ref[...],
                                    (((1,), (0,)), ((), ())),
                                    preferred_element_type=jnp.float32)

    e2 = jnp.exp(_INV_T * m)
    s2 = jnp.sum(e2, axis=0, keepdims=True)
    assign_ref[...] = (e2 / s2)[None]

    e1 = jnp.exp(m)
    qs = lax.dot_general(cnta_ref[0:65, :], e1, (((1,), (0,)), ((), ())),
                         preferred_element_type=jnp.float32)
    q_ref[...] = (qs[0:64, :] / qs[64:65, :])[None]


def kernel(x, codebook, cur_iter):
    del cur_iter
    xf = x.reshape(_B, _C, _HW)
    cbt = jnp.transpose(codebook)

    dist, assign_t, q_t = pl.pallas_call(
        _vq_body,
        grid=(_N // _BN,),
        in_specs=[
            pl.BlockSpec((1, _C, _BN), lambda i: (i // _NPB, 0, i % _NPB)),
            pl.BlockSpec((_C, _K), lambda i: (0, 0)),
        ],
        out_specs=[
            pl.BlockSpec((_BN, _K), lambda i: (i, 0)),
            pl.BlockSpec((1, _K, _BN), lambda i: (i // _NPB, 0, i % _NPB)),
            pl.BlockSpec((1, _C, _BN), lambda i: (i // _NPB, 0, i % _NPB)),
        ],
        out_shape=[
            jax.ShapeDtypeStruct((_N, _K), jnp.float32),
            jax.ShapeDtypeStruct((_B, _K, _HW), jnp.float32),
            jax.ShapeDtypeStruct((_B, _C, _HW), jnp.float32),
        ],
        scratch_shapes=[
            pltpu.VMEM((66, _K), jnp.float32),
        ],
    )(xf, cbt)

    q_feat = q_t.reshape(_B, _C, _H, _W)
    assignment = assign_t.reshape(_B, _K, _H, _W)
    return q_feat, assignment, dist
